# Initial kernel scaffold; baseline (speedup 1.0000x reference)
import jax
import jax.numpy as jnp
from jax.experimental import pallas as pl


def kernel(x):
    raise NotImplementedError("write your pallas kernel here")


# speedup vs baseline: 1.9270x; 1.9270x over previous
import functools

import jax
import jax.numpy as jnp
import numpy as np
from jax import lax
from jax.experimental import pallas as pl
from jax.experimental.pallas import tpu as pltpu
from jax.experimental.pallas import tpu_sc as plsc

_H = 64
_L = _H * _H
_ROWS = 16 * 96
_NW = 32
_ROWS_PER_W = _ROWS // _NW
_CHUNK = 8
_NCHUNK = _ROWS_PER_W // _CHUNK
_GROUPS = _L // 16


def _inverse_morton(H: int) -> np.ndarray:
    l = H * H
    bit_l = l.bit_length()
    i = np.arange(H, dtype=np.int64)[:, None]
    j = np.arange(H, dtype=np.int64)[None, :]
    ij = np.zeros((H, H), dtype=np.int64)
    for bit in range(bit_l):
        if bit % 2 == 1:
            ij = ij + (((i >> (bit // 2)) & 1) << bit)
        else:
            ij = ij + (((j >> (bit // 2)) & 1) << bit)
    ij = ij.reshape(-1)
    inv = np.empty(l, dtype=np.int32)
    inv[ij] = np.arange(l, dtype=np.int32)
    return inv


def _body(x_hbm, inv_hbm, out_hbm, inv_v, in_v, out_v):
    nc = 2
    wid = lax.axis_index("s") * nc + lax.axis_index("c")
    pltpu.sync_copy(inv_hbm, inv_v)
    elem0 = wid * _ROWS_PER_W * _L

    def chunk_body(ci, _):
        base = elem0 + ci * _CHUNK * _L
        pltpu.sync_copy(x_hbm.at[pl.ds(base, _CHUNK * _L)], in_v)

        def row_body(r, _):
            rbase = jnp.full((16,), r * _L, dtype=jnp.int32)

            def grp_body(g, _):
                for k in range(8):
                    off = (g * 8 + k) * 16
                    idx = inv_v[pl.ds(off, 16)] + rbase
                    vals = plsc.load_gather(in_v, [idx])
                    out_v[pl.ds(r * _L + off, 16)] = vals
                return 0

            lax.fori_loop(0, _GROUPS // 8, grp_body, 0, unroll=False)
            return 0

        lax.fori_loop(0, _CHUNK, row_body, 0, unroll=False)
        pltpu.sync_copy(out_v, out_hbm.at[pl.ds(base, _CHUNK * _L)])
        return 0

    lax.fori_loop(0, _NCHUNK, chunk_body, 0, unroll=False)


@functools.partial(jax.jit, static_argnames=())
def kernel(x):
    B, C, H, _ = x.shape
    flat = x.reshape(B * C * H * H)
    inv = jnp.asarray(_inverse_morton(H))
    run = pl.kernel(
        _body,
        out_type=jax.ShapeDtypeStruct((B * C * H * H,), jnp.float32),
        mesh=plsc.VectorSubcoreMesh(core_axis_name="c", subcore_axis_name="s"),
        compiler_params=pltpu.CompilerParams(needs_layout_passes=False),
        scratch_types=[
            pltpu.VMEM((_L,), jnp.int32),
            pltpu.VMEM((_CHUNK * _L,), jnp.float32),
            pltpu.VMEM((_CHUNK * _L,), jnp.float32),
        ],
    )
    out = run(flat, inv)
    return out.reshape(B, C, H * H)


# speedup vs baseline: 3.5380x; 1.8360x over previous
import functools

import jax
import jax.numpy as jnp
import numpy as np
from jax import lax
from jax.experimental import pallas as pl
from jax.experimental.pallas import tpu as pltpu
from jax.experimental.pallas import tpu_sc as plsc

_H = 64
_L = _H * _H
_ROWS = 16 * 96
_NW = 32
_ROWS_PER_W = _ROWS // _NW
_CHUNK = 8
_NCHUNK = _ROWS_PER_W // _CHUNK
_GROUPS = _L // 16


def _inverse_morton(H: int) -> np.ndarray:
    l = H * H
    bit_l = l.bit_length()
    i = np.arange(H, dtype=np.int64)[:, None]
    j = np.arange(H, dtype=np.int64)[None, :]
    ij = np.zeros((H, H), dtype=np.int64)
    for bit in range(bit_l):
        if bit % 2 == 1:
            ij = ij + (((i >> (bit // 2)) & 1) << bit)
        else:
            ij = ij + (((j >> (bit // 2)) & 1) << bit)
    ij = ij.reshape(-1)
    inv = np.empty(l, dtype=np.int32)
    inv[ij] = np.arange(l, dtype=np.int32)
    return inv


def _body(x_hbm, inv_hbm, out_hbm, inv_v, in_v, out_v):
    nc = 2
    wid = lax.axis_index("s") * nc + lax.axis_index("c")
    pltpu.sync_copy(inv_hbm, inv_v)
    elem0 = wid * _ROWS_PER_W * _L

    rbases = [jnp.full((16,), r * _L, dtype=jnp.int32) for r in range(_CHUNK)]

    def chunk_body(ci, _):
        base = elem0 + ci * _CHUNK * _L
        pltpu.sync_copy(x_hbm.at[pl.ds(base, _CHUNK * _L)], in_v)

        @plsc.parallel_loop(0, _GROUPS, step=1, unroll=4)
        def grp_body(g):
            off = g * 16
            idx = inv_v[pl.ds(off, 16)]
            for r in range(_CHUNK):
                vals = plsc.load_gather(in_v, [idx + rbases[r]])
                out_v[pl.ds(r * _L + off, 16)] = vals

        pltpu.sync_copy(out_v, out_hbm.at[pl.ds(base, _CHUNK * _L)])
        return 0

    lax.fori_loop(0, _NCHUNK, chunk_body, 0, unroll=False)


@functools.partial(jax.jit, static_argnames=())
def kernel(x):
    B, C, H, _ = x.shape
    flat = x.reshape(B * C * H * H)
    inv = jnp.asarray(_inverse_morton(H))
    run = pl.kernel(
        _body,
        out_type=jax.ShapeDtypeStruct((B * C * H * H,), jnp.float32),
        mesh=plsc.VectorSubcoreMesh(core_axis_name="c", subcore_axis_name="s"),
        compiler_params=pltpu.CompilerParams(needs_layout_passes=False),
        scratch_types=[
            pltpu.VMEM((_L,), jnp.int32),
            pltpu.VMEM((_CHUNK * _L,), jnp.float32),
            pltpu.VMEM((_CHUNK * _L,), jnp.float32),
        ],
    )
    out = run(flat, inv)
    return out.reshape(B, C, H * H)


# speedup vs baseline: 5.6963x; 1.6100x over previous
import functools

import jax
import jax.numpy as jnp
import numpy as np
from jax import lax
from jax.experimental import pallas as pl
from jax.experimental.pallas import tpu as pltpu
from jax.experimental.pallas import tpu_sc as plsc

_H = 64
_L = _H * _H
_ROWS = 16 * 96
_NW = 32
_ROWS_PER_W = _ROWS // _NW
_CHUNK = 8
_NCHUNK = _ROWS_PER_W // _CHUNK
_GROUPS = _L // 16


def _inverse_morton(H: int) -> np.ndarray:
    l = H * H
    bit_l = l.bit_length()
    i = np.arange(H, dtype=np.int64)[:, None]
    j = np.arange(H, dtype=np.int64)[None, :]
    ij = np.zeros((H, H), dtype=np.int64)
    for bit in range(bit_l):
        if bit % 2 == 1:
            ij = ij + (((i >> (bit // 2)) & 1) << bit)
        else:
            ij = ij + (((j >> (bit // 2)) & 1) << bit)
    ij = ij.reshape(-1)
    inv = np.empty(l, dtype=np.int32)
    inv[ij] = np.arange(l, dtype=np.int32)
    return inv


def _body(x_hbm, inv_hbm, out_hbm, inv_v, in_v, out_v):
    nc = 2
    wid = lax.axis_index("s") * nc + lax.axis_index("c")
    pltpu.sync_copy(inv_hbm, inv_v)
    row0 = wid * _ROWS_PER_W

    rsplats = [jnp.full((16,), r, dtype=jnp.int32) for r in range(_CHUNK)]

    def chunk_body(ci, _):
        base = row0 + ci * _CHUNK
        pltpu.sync_copy(x_hbm.at[pl.ds(base, _CHUNK)], in_v)

        @plsc.parallel_loop(0, _GROUPS, step=1, unroll=4)
        def grp_body(g):
            off = g * 16
            idx = inv_v[pl.ds(off, 16)]
            ii = idx >> 6
            ij = idx & 63
            for r in range(_CHUNK):
                vals = plsc.load_gather(in_v, [rsplats[r], ii, ij])
                out_v[r, pl.ds(off, 16)] = vals

        pltpu.sync_copy(out_v, out_hbm.at[pl.ds(base, _CHUNK)])
        return 0

    lax.fori_loop(0, _NCHUNK, chunk_body, 0, unroll=False)


@functools.partial(jax.jit, static_argnames=())
def kernel(x):
    B, C, H, _ = x.shape
    xv = x.reshape(B * C, H, H)
    inv = jnp.asarray(_inverse_morton(H))
    run = pl.kernel(
        _body,
        out_type=jax.ShapeDtypeStruct((B * C, H * H), jnp.float32),
        mesh=plsc.VectorSubcoreMesh(core_axis_name="c", subcore_axis_name="s"),
        compiler_params=pltpu.CompilerParams(needs_layout_passes=False),
        scratch_types=[
            pltpu.VMEM((_L,), jnp.int32),
            pltpu.VMEM((_CHUNK, _H, _H), jnp.float32),
            pltpu.VMEM((_CHUNK, _L), jnp.float32),
        ],
    )
    out = run(xv, inv)
    return out.reshape(B, C, H * H)
